# baseline (speedup 1.0000x reference)
import functools

import jax
import jax.numpy as jnp
from jax import lax
from jax.experimental import pallas as pl
from jax.experimental.pallas import tpu as pltpu
from jax.experimental.pallas import tpu_sc as plsc

_B = 16384
_D = 16
_NC = 2
_NS = 16
_NW = _NC * _NS
_BPW = _B // _NW
_C = 128
_G = 16


def _body(uid_hbm, cid_hbm, wd_hbm, tf_hbm,
          ut_hbm, ct_hbm, wt_hbm, tt_hbm,
          out_hbm,
          iu, ic, iw, it, ubuf, cbuf, wbuf, tbuf, out_v, sem, gsem):
    wid = lax.axis_index("s") * _NC + lax.axis_index("c")
    base = wid * _BPW

    pltpu.sync_copy(uid_hbm.at[pl.ds(base, _BPW)], iu)
    pltpu.sync_copy(cid_hbm.at[pl.ds(base, _BPW)], ic)
    pltpu.sync_copy(wd_hbm.at[pl.ds(base, _BPW)], iw)
    pltpu.sync_copy(tf_hbm.at[pl.ds(base, _BPW)], it)
    cp_w = pltpu.async_copy(wt_hbm, wbuf, sem)
    cp_t = pltpu.async_copy(tt_hbm, tbuf, sem)
    cp_w.wait()
    cp_t.wait()

    def chunk(k, _):
        off = k * _C

        def fire(g, _):
            uvec = iu[pl.ds(off + g * _G, _G)]
            cvec = ic[pl.ds(off + g * _G, _G)]
            for j in range(_G):
                row = g * _G + j
                pltpu.async_copy(ut_hbm.at[pl.ds(uvec[j], 1), :],
                                 ubuf.at[pl.ds(row, 1), :], gsem)
                pltpu.async_copy(ct_hbm.at[pl.ds(cvec[j], 1), :],
                                 cbuf.at[pl.ds(row, 1), :], gsem)
            return 0

        lax.fori_loop(0, _C // _G, fire, 0)

        def small(g, _):
            wvec = iw[pl.ds(off + g * _G, _G)]
            tvec = it[pl.ds(off + g * _G, _G)]
            for j in range(_G):
                row = g * _G + j
                out_v[row, pl.ds(2 * _D, _D)] = wbuf[wvec[j], :]
                out_v[row, pl.ds(3 * _D, _D)] = tbuf[tvec[j], :]
            return 0

        lax.fori_loop(0, _C // _G, small, 0)

        pltpu.make_async_copy(ut_hbm.at[pl.ds(0, _C), :], ubuf, gsem).wait()
        pltpu.make_async_copy(ct_hbm.at[pl.ds(0, _C), :], cbuf, gsem).wait()

        def assemble(g, _):
            for j in range(_G):
                row = g * _G + j
                out_v[row, pl.ds(0 * _D, _D)] = ubuf[row, :]
                out_v[row, pl.ds(1 * _D, _D)] = cbuf[row, :]
            return 0

        lax.fori_loop(0, _C // _G, assemble, 0)

        pltpu.sync_copy(out_v, out_hbm.at[pl.ds(base + off, _C), :])
        return 0

    lax.fori_loop(0, _BPW // _C, chunk, 0)


@jax.jit
def _run(uid, cid, wd, tf, ut, ct, wt, tt):
    mesh = plsc.VectorSubcoreMesh(core_axis_name="c", subcore_axis_name="s")
    return pl.kernel(
        _body,
        out_type=jax.ShapeDtypeStruct((_B, 4 * _D), jnp.float32),
        mesh=mesh,
        scratch_types=[
            pltpu.VMEM((_BPW,), jnp.int32),
            pltpu.VMEM((_BPW,), jnp.int32),
            pltpu.VMEM((_BPW,), jnp.int32),
            pltpu.VMEM((_BPW,), jnp.int32),
            pltpu.VMEM((_C, _D), jnp.float32),
            pltpu.VMEM((_C, _D), jnp.float32),
            pltpu.VMEM((8, _D), jnp.float32),
            pltpu.VMEM((25, _D), jnp.float32),
            pltpu.VMEM((_C, 4 * _D), jnp.float32),
            pltpu.SemaphoreType.DMA,
            pltpu.SemaphoreType.DMA,
        ],
    )(uid, cid, wd, tf, ut, ct, wt, tt)


def kernel(user_id, category_id, weekday, time_frames,
           user_table, category_table, weekday_table, time_frame_table):
    return _run(user_id.astype(jnp.int32), category_id.astype(jnp.int32),
                weekday.astype(jnp.int32), time_frames.astype(jnp.int32),
                user_table, category_table, weekday_table, time_frame_table)
